# baseline (speedup 1.0000x reference)
import jax
import jax.numpy as jnp
from jax.experimental import pallas as pl
from jax.experimental.pallas import tpu as pltpu

B, N, D_FEAT = 4, 512, 128
N0 = N - 1
N1 = N // 2
BIG = 1e30
IDX_MASK = N - 1


def _betti_kernel(x_ref, out_ref, dm_ref):
    ri = jax.lax.broadcasted_iota(jnp.int32, (N, N), 0)
    ci = jax.lax.broadcasted_iota(jnp.int32, (N, N), 1)

    for b in range(B):
        pts = x_ref[b]
        nrm = jnp.sqrt(jnp.sum(pts * pts, axis=0, keepdims=True))
        pts = pts / jnp.maximum(nrm, 1e-12)
        g = jax.lax.dot_general(
            pts, pts, (((1,), (1,)), ((), ())),
            preferred_element_type=jnp.float32,
            precision=jax.lax.Precision.DEFAULT,
        )
        sq = jnp.sum(pts * pts, axis=1, keepdims=True)
        sqc = jnp.min(jnp.where(ri == ci, g, BIG), axis=0, keepdims=True)
        d2 = jnp.maximum(sq + sqc - 2.0 * g, 0.0)
        dm = jnp.sqrt(d2 + 1e-12)
        dm = jnp.where(ri == ci, BIG, dm)
        kd = (jax.lax.bitcast_convert_type(dm, jnp.int32) & ~IDX_MASK) | ci
        kdf = jax.lax.bitcast_convert_type(kd, jnp.float32)
        dm_ref[b, :, :, :] = kdf.reshape(N, 4, 128)

    maxf = jnp.float32(jnp.finfo(jnp.float32).max)

    def step(t, carry):
        mds, accs, nns = carry
        new_mds, new_accs, new_nns = [], [], []
        for b in range(B):
            kmin = jnp.min(mds[b])
            j = jax.lax.bitcast_convert_type(kmin, jnp.int32) & IDX_MASK
            row = dm_ref[b, pl.ds(j, 1), :, :].reshape(4, 128)
            new_mds.append(
                jnp.where(mds[b] == kmin, maxf, jnp.minimum(mds[b], row))
            )
            new_accs.append(accs[b] + kmin * kmin)
            rowt = dm_ref[b, pl.ds(t, 1), :, :].reshape(4, 128)
            m1 = jnp.min(rowt)
            m2 = jnp.min(jnp.where(rowt == m1, maxf, rowt))
            d = m2 - m1
            new_nns.append(
                nns[b] + jnp.where(t < N1, d * d, jnp.float32(0.0))
            )
        return new_mds, new_accs, new_nns

    md0 = [dm_ref[b, 0, :, :] for b in range(B)]
    acc0 = [jnp.float32(0.0) for _ in range(B)]
    nn0 = [jnp.float32(0.0) for _ in range(B)]
    _, accs, nns = jax.lax.fori_loop(
        0, N - 1, step, (md0, acc0, nn0), unroll=8
    )

    for b in range(B):
        out_ref[b] = jnp.float32(N0 + N1) - accs[b] - nns[b]


@jax.jit
def kernel(X, Y):
    del Y
    return pl.pallas_call(
        _betti_kernel,
        out_shape=jax.ShapeDtypeStruct((B,), jnp.float32),
        out_specs=pl.BlockSpec(memory_space=pltpu.SMEM),
        scratch_shapes=[pltpu.VMEM((B, N, 4, 128), jnp.float32)],
    )(X)


# speedup vs baseline: 3.7824x; 3.7824x over previous
import jax
import jax.numpy as jnp
from jax.experimental import pallas as pl
from jax.experimental.pallas import tpu as pltpu

B, N, D_FEAT = 4, 512, 128
N0 = N - 1
N1 = N // 2
BIG = 1e30
IDX_MASK = N - 1


def _betti_kernel(x_ref, out_ref, dm_ref):
    ri = jax.lax.broadcasted_iota(jnp.int32, (N, N), 0)
    ci = jax.lax.broadcasted_iota(jnp.int32, (N, N), 1)

    nn_vals = []
    for b in range(B):
        pts = x_ref[b]
        nrm = jnp.sqrt(jnp.sum(pts * pts, axis=0, keepdims=True))
        pts = pts / jnp.maximum(nrm, 1e-12)
        g = jax.lax.dot_general(
            pts, pts, (((1,), (1,)), ((), ())),
            preferred_element_type=jnp.float32,
            precision=jax.lax.Precision.DEFAULT,
        )
        sq = jnp.sum(pts * pts, axis=1, keepdims=True)
        sqc = jnp.min(jnp.where(ri == ci, g, BIG), axis=0, keepdims=True)
        d2 = jnp.maximum(sq + sqc - 2.0 * g, 0.0)
        dm = jnp.sqrt(d2 + 1e-12)
        dm = jnp.where(ri == ci, BIG, dm)
        kd = (jax.lax.bitcast_convert_type(dm, jnp.int32) & ~IDX_MASK) | ci
        kdf = jax.lax.bitcast_convert_type(kd, jnp.float32)
        dm_ref[b, :, :, :] = kdf.reshape(N, 4, 128)

        m1 = jnp.min(dm, axis=1, keepdims=True)
        jmin = jnp.min(jnp.where(dm == m1, ci, N), axis=1, keepdims=True)
        m2 = jnp.min(jnp.where(ci == jmin, BIG, dm), axis=1, keepdims=True)
        diff = m2 - m1
        rrow = jax.lax.broadcasted_iota(jnp.int32, (N, 1), 0)
        nn_vals.append(jnp.sum(jnp.where(rrow < N1, diff * diff, 0.0)))

    maxf = jnp.float32(jnp.finfo(jnp.float32).max)

    def step(_, carry):
        mds, accs = carry
        new_mds, new_accs = [], []
        for b in range(B):
            kmin = jnp.min(mds[b])
            j = jax.lax.bitcast_convert_type(kmin, jnp.int32) & IDX_MASK
            row = dm_ref[b, pl.ds(j, 1), :, :].reshape(4, 128)
            new_mds.append(
                jnp.where(mds[b] == kmin, maxf, jnp.minimum(mds[b], row))
            )
            new_accs.append(accs[b] + kmin * kmin)
        return new_mds, new_accs

    md0 = [dm_ref[b, 0, :, :] for b in range(B)]
    acc0 = [jnp.float32(0.0) for _ in range(B)]
    _, accs = jax.lax.fori_loop(0, N - 1, step, (md0, acc0), unroll=8)

    for b in range(B):
        out_ref[b] = jnp.float32(N0 + N1) - accs[b] - nn_vals[b]


@jax.jit
def kernel(X, Y):
    del Y
    return pl.pallas_call(
        _betti_kernel,
        out_shape=jax.ShapeDtypeStruct((B,), jnp.float32),
        out_specs=pl.BlockSpec(memory_space=pltpu.SMEM),
        scratch_shapes=[pltpu.VMEM((B, N, 4, 128), jnp.float32)],
    )(X)
